# baseline (speedup 1.0000x reference)
import functools

import jax
import jax.numpy as jnp
from jax import lax
from jax.experimental import pallas as pl
from jax.experimental.pallas import tpu as pltpu
from jax.experimental.pallas import tpu_sc as plsc

_B = 16384
_D = 64
_NC = 2
_NS = 16
_NW = _NC * _NS
_BPW = _B // _NW
_CH = 32
_NCHK = _BPW // _CH
_LANES = 16
_PK = 4
_PW = _D // 2
_HI = -65536


def _pack_table(table):
    n = table.shape[0]
    t16 = table.astype(jnp.bfloat16)
    t16 = t16.reshape(n, 2, _PW).transpose(0, 2, 1)
    pk = lax.bitcast_convert_type(t16, jnp.int32)
    return pk.reshape(n // _PK, _PK * _PW)


def _transe_body(ent_hbm, rel_hbm, h_hbm, r_hbm, t_hbm, out_hbm,
                 hv, rv, tv, gh, gr, gt, ob, sem):
    wid = lax.axis_index("s") * _NC + lax.axis_index("c")
    base = wid * _BPW

    pltpu.sync_copy(h_hbm.at[pl.ds(base, _BPW)], hv)
    pltpu.sync_copy(r_hbm.at[pl.ds(base, _BPW)], rv)
    pltpu.sync_copy(t_hbm.at[pl.ds(base, _BPW)], tv)

    lanes = lax.iota(jnp.int32, _LANES)
    zero = jnp.zeros((_LANES,), jnp.int32)
    masks = [lanes == l for l in range(_LANES)]

    def unpack(w):
        lo = lax.bitcast_convert_type(lax.shift_left(w, 16), jnp.float32)
        hi = lax.bitcast_convert_type(w & jnp.full((_LANES,), _HI, jnp.int32),
                                      jnp.float32)
        return lo, hi

    def chunk_step(j, carry):
        co = j * _CH
        cps = []
        qs = []
        for g in range(_CH // _LANES):
            s = pl.ds(co + g * _LANES, _LANES)
            hvv = hv[s]
            rvv = rv[s]
            tvv = tv[s]
            for l in range(_LANES):
                he = jnp.max(jnp.where(masks[l], hvv, zero))
                re_ = jnp.max(jnp.where(masks[l], rvv, zero))
                te = jnp.max(jnp.where(masks[l], tvv, zero))
                e = g * _LANES + l
                cps.append(pltpu.async_copy(ent_hbm.at[pl.ds(he >> 2, 1)],
                                            gh.at[pl.ds(e, 1)], sem))
                cps.append(pltpu.async_copy(rel_hbm.at[pl.ds(re_ >> 2, 1)],
                                            gr.at[pl.ds(e, 1)], sem))
                cps.append(pltpu.async_copy(ent_hbm.at[pl.ds(te >> 2, 1)],
                                            gt.at[pl.ds(e, 1)], sem))
                qs.append(((he & 3) * _PW, (re_ & 3) * _PW, (te & 3) * _PW))
        for cp in cps:
            cp.wait()
        for e in range(_CH):
            qh, qr, qt = qs[e]
            for c in range(_PW // _LANES):
                wh = gh[e, pl.ds(qh + c * _LANES, _LANES)]
                wr = gr[e, pl.ds(qr + c * _LANES, _LANES)]
                wt = gt[e, pl.ds(qt + c * _LANES, _LANES)]
                hlo, hhi = unpack(wh)
                rlo, rhi = unpack(wr)
                tlo, thi = unpack(wt)
                ob[e, pl.ds(c * _LANES, _LANES)] = hlo + rlo - tlo
                ob[e, pl.ds(_PW + c * _LANES, _LANES)] = hhi + rhi - thi
        pltpu.sync_copy(ob, out_hbm.at[pl.ds(base + co, _CH)])
        return carry
    lax.fori_loop(0, _NCHK, chunk_step, 0)


@functools.partial(
    pl.kernel,
    out_type=jax.ShapeDtypeStruct((_B, _D), jnp.float32),
    mesh=plsc.VectorSubcoreMesh(core_axis_name="c", subcore_axis_name="s"),
    compiler_params=pltpu.CompilerParams(needs_layout_passes=False),
    scratch_types=[
        pltpu.VMEM((_BPW,), jnp.int32),
        pltpu.VMEM((_BPW,), jnp.int32),
        pltpu.VMEM((_BPW,), jnp.int32),
        pltpu.VMEM((_CH, _PK * _PW), jnp.int32),
        pltpu.VMEM((_CH, _PK * _PW), jnp.int32),
        pltpu.VMEM((_CH, _PK * _PW), jnp.int32),
        pltpu.VMEM((_CH, _D), jnp.float32),
        pltpu.SemaphoreType.DMA,
    ],
)
def _transe(ent_hbm, rel_hbm, h_hbm, r_hbm, t_hbm, out_hbm,
            hv, rv, tv, gh, gr, gt, ob, sem):
    _transe_body(ent_hbm, rel_hbm, h_hbm, r_hbm, t_hbm, out_hbm,
                 hv, rv, tv, gh, gr, gt, ob, sem)


def kernel(h, r, t, entity_embeddings, relation_embeddings):
    ent_pk = _pack_table(entity_embeddings)
    rel_pk = _pack_table(relation_embeddings)
    return _transe(ent_pk, rel_pk, h, r, t)


# speedup vs baseline: 1.1173x; 1.1173x over previous
import functools

import jax
import jax.numpy as jnp
from jax import lax
from jax.experimental import pallas as pl
from jax.experimental.pallas import tpu as pltpu
from jax.experimental.pallas import tpu_sc as plsc

_B = 16384
_D = 64
_NC = 2
_NS = 16
_NW = _NC * _NS
_BPW = _B // _NW
_CH = 32
_NCHK = _BPW // _CH
_LANES = 16
_PK = 4
_PW = _D // 2
_HI = -65536


def _pack_table(table):
    n = table.shape[0]
    bits = lax.bitcast_convert_type(table, jnp.uint32)
    rnd = jnp.uint32(0x8000)
    wlo = lax.shift_right_logical(bits[:, :_PW] + rnd, jnp.uint32(16))
    whi = (bits[:, _PW:] + rnd) & jnp.uint32(0xFFFF0000)
    w = lax.bitcast_convert_type(wlo | whi, jnp.int32)
    return w.reshape(n // _PK, _PK * _PW)


def _transe_body(ent_hbm, rel_hbm, h_hbm, r_hbm, t_hbm, out_hbm,
                 hv, rv, tv, ph, pr, pt, gh, gr, gt, ob, sem):
    wid = lax.axis_index("s") * _NC + lax.axis_index("c")
    base = wid * _BPW

    pltpu.sync_copy(h_hbm.at[pl.ds(base, _BPW)], hv)
    pltpu.sync_copy(r_hbm.at[pl.ds(base, _BPW)], rv)
    pltpu.sync_copy(t_hbm.at[pl.ds(base, _BPW)], tv)

    grp_per_row = _CH // _LANES
    for k in range(_BPW // _LANES):
        s = pl.ds(k * _LANES, _LANES)
        d0 = k // grp_per_row
        d1 = pl.ds((k % grp_per_row) * _LANES, _LANES)
        ph[d0, d1] = lax.shift_right_logical(hv[s], 2)
        pr[d0, d1] = lax.shift_right_logical(rv[s], 2)
        pt[d0, d1] = lax.shift_right_logical(tv[s], 2)

    lanes = lax.iota(jnp.int32, _LANES)
    zero = jnp.zeros((_LANES,), jnp.int32)
    masks = [lanes == l for l in range(_LANES)]
    himask = jnp.full((_LANES,), _HI, jnp.int32)

    def unpack(w):
        lo = lax.bitcast_convert_type(lax.shift_left(w, 16), jnp.float32)
        hi = lax.bitcast_convert_type(w & himask, jnp.float32)
        return lo, hi

    def chunk_step(j, carry):
        co = j * _CH
        cp_h = pltpu.async_copy(ent_hbm.at[ph.at[j]], gh, sem)
        cp_r = pltpu.async_copy(rel_hbm.at[pr.at[j]], gr, sem)
        cp_t = pltpu.async_copy(ent_hbm.at[pt.at[j]], gt, sem)
        cp_h.wait()
        cp_r.wait()
        cp_t.wait()
        for g in range(_CH // _LANES):
            sl = pl.ds(co + g * _LANES, _LANES)
            qh = hv[sl]
            qr = rv[sl]
            qt = tv[sl]
            for l in range(_LANES):
                oh = (jnp.max(jnp.where(masks[l], qh, zero)) & 3) * _PW
                or_ = (jnp.max(jnp.where(masks[l], qr, zero)) & 3) * _PW
                ot = (jnp.max(jnp.where(masks[l], qt, zero)) & 3) * _PW
                e = g * _LANES + l
                for c in range(_PW // _LANES):
                    wh = gh[e, pl.ds(oh + c * _LANES, _LANES)]
                    wr = gr[e, pl.ds(or_ + c * _LANES, _LANES)]
                    wt = gt[e, pl.ds(ot + c * _LANES, _LANES)]
                    hlo, hhi = unpack(wh)
                    rlo, rhi = unpack(wr)
                    tlo, thi = unpack(wt)
                    ob[e, pl.ds(c * _LANES, _LANES)] = hlo + rlo - tlo
                    ob[e, pl.ds(_PW + c * _LANES, _LANES)] = hhi + rhi - thi
        pltpu.sync_copy(ob, out_hbm.at[pl.ds(base + co, _CH)])
        return carry
    lax.fori_loop(0, _NCHK, chunk_step, 0)


@functools.partial(
    pl.kernel,
    out_type=jax.ShapeDtypeStruct((_B, _D), jnp.float32),
    mesh=plsc.VectorSubcoreMesh(core_axis_name="c", subcore_axis_name="s"),
    compiler_params=pltpu.CompilerParams(needs_layout_passes=False),
    scratch_types=[
        pltpu.VMEM((_BPW,), jnp.int32),
        pltpu.VMEM((_BPW,), jnp.int32),
        pltpu.VMEM((_BPW,), jnp.int32),
        pltpu.VMEM((_NCHK, _CH), jnp.int32),
        pltpu.VMEM((_NCHK, _CH), jnp.int32),
        pltpu.VMEM((_NCHK, _CH), jnp.int32),
        pltpu.VMEM((_CH, _PK * _PW), jnp.int32),
        pltpu.VMEM((_CH, _PK * _PW), jnp.int32),
        pltpu.VMEM((_CH, _PK * _PW), jnp.int32),
        pltpu.VMEM((_CH, _D), jnp.float32),
        pltpu.SemaphoreType.DMA,
    ],
)
def _transe(ent_hbm, rel_hbm, h_hbm, r_hbm, t_hbm, out_hbm,
            hv, rv, tv, ph, pr, pt, gh, gr, gt, ob, sem):
    _transe_body(ent_hbm, rel_hbm, h_hbm, r_hbm, t_hbm, out_hbm,
                 hv, rv, tv, ph, pr, pt, gh, gr, gt, ob, sem)


def kernel(h, r, t, entity_embeddings, relation_embeddings):
    ent_pk = _pack_table(entity_embeddings)
    rel_pk = _pack_table(relation_embeddings)
    return _transe(ent_pk, rel_pk, h, r, t)


# speedup vs baseline: 1.3748x; 1.2305x over previous
import functools

import jax
import jax.numpy as jnp
from jax import lax
from jax.experimental import pallas as pl
from jax.experimental.pallas import tpu as pltpu
from jax.experimental.pallas import tpu_sc as plsc

_B = 16384
_D = 64
_NC = 2
_NS = 16
_NW = _NC * _NS
_BPW = _B // _NW
_CH = 64
_NCHK = _BPW // _CH
_LANES = 16
_PK = 2


def _transe_body(ent_hbm, rel_hbm, h_hbm, r_hbm, t_hbm, out_hbm,
                 hv, rv, tv, ph, pr, pt, gh, gr, gt, ob, sem):
    wid = lax.axis_index("s") * _NC + lax.axis_index("c")
    base = wid * _BPW

    pltpu.sync_copy(h_hbm.at[pl.ds(base, _BPW)], hv)
    pltpu.sync_copy(r_hbm.at[pl.ds(base, _BPW)], rv)
    pltpu.sync_copy(t_hbm.at[pl.ds(base, _BPW)], tv)

    grp_per_row = _CH // _LANES
    for k in range(_BPW // _LANES):
        s = pl.ds(k * _LANES, _LANES)
        d0 = k // grp_per_row
        d1 = pl.ds((k % grp_per_row) * _LANES, _LANES)
        ph[d0, d1] = lax.shift_right_logical(hv[s], 1)
        pr[d0, d1] = lax.shift_right_logical(rv[s], 1)
        pt[d0, d1] = lax.shift_right_logical(tv[s], 1)

    lanes = lax.iota(jnp.int32, _LANES)
    zero = jnp.zeros((_LANES,), jnp.int32)
    masks = [lanes == l for l in range(_LANES)]

    def chunk_step(j, carry):
        co = j * _CH
        cp_h = pltpu.async_copy(ent_hbm.at[ph.at[j]], gh, sem)
        cp_r = pltpu.async_copy(rel_hbm.at[pr.at[j]], gr, sem)
        cp_t = pltpu.async_copy(ent_hbm.at[pt.at[j]], gt, sem)
        cp_h.wait()
        cp_r.wait()
        cp_t.wait()
        for g in range(_CH // _LANES):
            sl = pl.ds(co + g * _LANES, _LANES)
            qh = hv[sl]
            qr = rv[sl]
            qt = tv[sl]
            for l in range(_LANES):
                oh = (jnp.max(jnp.where(masks[l], qh, zero)) & 1) * _D
                or_ = (jnp.max(jnp.where(masks[l], qr, zero)) & 1) * _D
                ot = (jnp.max(jnp.where(masks[l], qt, zero)) & 1) * _D
                e = g * _LANES + l
                for c in range(_D // _LANES):
                    wh = gh[e, pl.ds(oh + c * _LANES, _LANES)]
                    wr = gr[e, pl.ds(or_ + c * _LANES, _LANES)]
                    wt = gt[e, pl.ds(ot + c * _LANES, _LANES)]
                    ob[e, pl.ds(c * _LANES, _LANES)] = wh + wr - wt
        pltpu.sync_copy(ob, out_hbm.at[pl.ds(base + co, _CH)])
        return carry
    lax.fori_loop(0, _NCHK, chunk_step, 0)


@functools.partial(
    pl.kernel,
    out_type=jax.ShapeDtypeStruct((_B, _D), jnp.float32),
    mesh=plsc.VectorSubcoreMesh(core_axis_name="c", subcore_axis_name="s"),
    compiler_params=pltpu.CompilerParams(needs_layout_passes=False),
    scratch_types=[
        pltpu.VMEM((_BPW,), jnp.int32),
        pltpu.VMEM((_BPW,), jnp.int32),
        pltpu.VMEM((_BPW,), jnp.int32),
        pltpu.VMEM((_NCHK, _CH), jnp.int32),
        pltpu.VMEM((_NCHK, _CH), jnp.int32),
        pltpu.VMEM((_NCHK, _CH), jnp.int32),
        pltpu.VMEM((_CH, _PK * _D), jnp.float32),
        pltpu.VMEM((_CH, _PK * _D), jnp.float32),
        pltpu.VMEM((_CH, _PK * _D), jnp.float32),
        pltpu.VMEM((_CH, _D), jnp.float32),
        pltpu.SemaphoreType.DMA,
    ],
)
def _transe(ent_hbm, rel_hbm, h_hbm, r_hbm, t_hbm, out_hbm,
            hv, rv, tv, ph, pr, pt, gh, gr, gt, ob, sem):
    _transe_body(ent_hbm, rel_hbm, h_hbm, r_hbm, t_hbm, out_hbm,
                 hv, rv, tv, ph, pr, pt, gh, gr, gt, ob, sem)


def kernel(h, r, t, entity_embeddings, relation_embeddings):
    ent2 = entity_embeddings.reshape(-1, _PK * _D)
    rel2 = relation_embeddings.reshape(-1, _PK * _D)
    return _transe(ent2, rel2, h, r, t)
